# baseline (speedup 1.0000x reference)
import functools

import jax
import jax.numpy as jnp
from jax import lax
from jax.experimental import pallas as pl
from jax.experimental.pallas import tpu as pltpu
from jax.experimental.pallas import tpu_sc as plsc

N_FIELDS = 26
N_PAIRS = N_FIELDS // 2
VALS_PER_FIELD = 32
OUTPUT_DIM = 64
BATCH = 16384

_info = plsc.get_sparse_core_info()
NC, NS, L = _info.num_cores, _info.num_subcores, _info.num_lanes
NW = NC * NS
BB = 256
GI = 128
ROWS_W = BATCH // NW
K_HALVES = ROWS_W // BB
PER_W = K_HALVES * N_PAIRS
NBUF = 3
SKEW = 1


@functools.partial(
    pl.kernel,
    mesh=plsc.VectorSubcoreMesh(core_axis_name="c", subcore_axis_name="s"),
    out_type=jax.ShapeDtypeStruct((BATCH, N_FIELDS * OUTPUT_DIM), jnp.float32),
    scratch_types=[
        pltpu.VMEM((N_FIELDS * ROWS_W,), jnp.int32),
        pltpu.VMEM((NBUF * BB,), jnp.int32),
        pltpu.VMEM((NBUF, BB, 2 * OUTPUT_DIM), jnp.float32),
        pltpu.SemaphoreType.DMA,
    ]
    + [pltpu.SemaphoreType.DMA] * (2 * NBUF),
    compiler_params=pltpu.CompilerParams(use_tc_tiling_on_sc=True),
)
def _sc_gather(inT_hbm, ptab_hbm, out_hbm, vall, idxbuf, rows, vsem, *sems):
    gsems = sems[:NBUF]
    ssems = sems[NBUF:]
    wid = lax.axis_index("s") * NC + lax.axis_index("c")
    row0 = wid * ROWS_W

    vh = [
        pltpu.async_copy(
            inT_hbm.at[pl.ds(f * BATCH + row0, ROWS_W)],
            vall.at[pl.ds(f * ROWS_W, ROWS_W)],
            vsem,
        )
        for f in range(N_FIELDS)
    ]

    ghandles = [None] * PER_W
    shandles = [None] * PER_W

    def start_gather(c):
        s = c % NBUF
        t, k = c % N_PAIRS, c // N_PAIRS
        for i in range(BB // L):
            v0 = vall[pl.ds(2 * t * ROWS_W + k * BB + i * L, L)]
            v1 = vall[pl.ds((2 * t + 1) * ROWS_W + k * BB + i * L, L)]
            c0 = jnp.where((v0 >= 0) & (v0 < VALS_PER_FIELD), v0, 0)
            c1 = jnp.where((v1 >= 0) & (v1 < VALS_PER_FIELD), v1, 0)
            idxbuf[pl.ds(s * BB + i * L, L)] = t * 1024 + c0 * VALS_PER_FIELD + c1
        h0 = pltpu.async_copy(
            ptab_hbm.at[idxbuf.at[pl.ds(s * BB, GI)]],
            rows.at[s, pl.ds(0, GI)],
            gsems[s],
        )
        h1 = pltpu.async_copy(
            ptab_hbm.at[idxbuf.at[pl.ds(s * BB + GI, GI)]],
            rows.at[s, pl.ds(GI, GI)],
            gsems[s],
        )
        ghandles[c] = (h0, h1)

    def start_out(c):
        s = c % NBUF
        t, k = c % N_PAIRS, c // N_PAIRS
        b0 = row0 + k * BB
        for h in ghandles[c]:
            h.wait()
        shandles[c] = pltpu.async_copy(
            rows.at[s],
            out_hbm.at[pl.ds(b0, BB), pl.ds(t * 2 * OUTPUT_DIM, 2 * OUTPUT_DIM)],
            ssems[s],
        )

    for h in vh:
        h.wait()

    for c in range(PER_W + SKEW):
        if c < PER_W:
            if c >= NBUF:
                shandles[c - NBUF].wait()
            start_gather(c)
        d = c - SKEW
        if 0 <= d < PER_W:
            start_out(d)
    for d in range(PER_W - NBUF, PER_W):
        shandles[d].wait()


def kernel(input_features, table):
    inT = input_features.T.reshape(N_FIELDS * BATCH)
    tbl3 = table.reshape(N_PAIRS, 2 * VALS_PER_FIELD, OUTPUT_DIM)
    left = jnp.broadcast_to(
        tbl3[:, :VALS_PER_FIELD, None, :],
        (N_PAIRS, VALS_PER_FIELD, VALS_PER_FIELD, OUTPUT_DIM),
    )
    right = jnp.broadcast_to(
        tbl3[:, None, VALS_PER_FIELD:, :],
        (N_PAIRS, VALS_PER_FIELD, VALS_PER_FIELD, OUTPUT_DIM),
    )
    ptab = jnp.concatenate([left, right], axis=3).reshape(
        N_PAIRS * VALS_PER_FIELD * VALS_PER_FIELD, 2 * OUTPUT_DIM
    )
    return _sc_gather(inT, ptab)


# speedup vs baseline: 1.0471x; 1.0471x over previous
import functools

import jax
import jax.numpy as jnp
from jax import lax
from jax.experimental import pallas as pl
from jax.experimental.pallas import tpu as pltpu
from jax.experimental.pallas import tpu_sc as plsc

N_FIELDS = 26
N_PAIRS = N_FIELDS // 2
VALS_PER_FIELD = 32
OUTPUT_DIM = 64
BATCH = 16384

_info = plsc.get_sparse_core_info()
NC, NS, L = _info.num_cores, _info.num_subcores, _info.num_lanes
NW = NC * NS
BB = 256
GI = 128
N_BCHUNK = BATCH // BB
TOT_CHUNKS = N_PAIRS * N_BCHUNK
PER_W = TOT_CHUNKS // NW
NBUF = 3
SKEW = 1


@functools.partial(
    pl.kernel,
    mesh=plsc.VectorSubcoreMesh(core_axis_name="c", subcore_axis_name="s"),
    out_type=jax.ShapeDtypeStruct((BATCH, N_FIELDS * OUTPUT_DIM), jnp.float32),
    scratch_types=[
        pltpu.VMEM((NBUF * 2 * BB,), jnp.int32),
        pltpu.VMEM((NBUF * BB,), jnp.int32),
        pltpu.VMEM((NBUF, BB, 2 * OUTPUT_DIM), jnp.float32),
    ]
    + [pltpu.SemaphoreType.DMA] * (3 * NBUF),
    compiler_params=pltpu.CompilerParams(use_tc_tiling_on_sc=True),
)
def _sc_gather(inT_hbm, ptab_hbm, out_hbm, vbuf, idxbuf, rows, *sems):
    vsems = sems[:NBUF]
    gsems = sems[NBUF : 2 * NBUF]
    ssems = sems[2 * NBUF :]
    wid = lax.axis_index("s") * NC + lax.axis_index("c")
    cid0 = wid * PER_W
    lane = lax.iota(jnp.int32, L)

    vhandles = [None] * PER_W
    ghandles = [None] * PER_W
    shandles = [None] * PER_W

    def chunk_coords(c):
        cid = cid0 + c
        t = cid // N_BCHUNK
        b0 = (cid % N_BCHUNK) * BB
        return t, b0

    def stage_v(c):
        s = c % NBUF
        t, b0 = chunk_coords(c)
        h0 = pltpu.async_copy(
            inT_hbm.at[pl.ds(2 * t * BATCH + b0, BB)],
            vbuf.at[pl.ds((s * 2) * BB, BB)],
            vsems[s],
        )
        h1 = pltpu.async_copy(
            inT_hbm.at[pl.ds((2 * t + 1) * BATCH + b0, BB)],
            vbuf.at[pl.ds((s * 2 + 1) * BB, BB)],
            vsems[s],
        )
        vhandles[c] = (h0, h1)

    def start_gather(c):
        s = c % NBUF
        t, _ = chunk_coords(c)
        for h in vhandles[c]:
            h.wait()
        for i in range(BB // L):
            v0 = vbuf[pl.ds((s * 2) * BB + i * L, L)]
            v1 = vbuf[pl.ds((s * 2 + 1) * BB + i * L, L)]
            c0 = jnp.where((v0 >= 0) & (v0 < VALS_PER_FIELD), v0, 0)
            c1 = jnp.where((v1 >= 0) & (v1 < VALS_PER_FIELD), v1, 0)
            idxbuf[pl.ds(s * BB + i * L, L)] = t * 1024 + c0 * VALS_PER_FIELD + c1
        h0 = pltpu.async_copy(
            ptab_hbm.at[idxbuf.at[pl.ds(s * BB, GI)]],
            rows.at[s, pl.ds(0, GI)],
            gsems[s],
        )
        h1 = pltpu.async_copy(
            ptab_hbm.at[idxbuf.at[pl.ds(s * BB + GI, GI)]],
            rows.at[s, pl.ds(GI, GI)],
            gsems[s],
        )
        ghandles[c] = (h0, h1)

    def start_out(c):
        s = c % NBUF
        t, b0 = chunk_coords(c)
        for h in ghandles[c]:
            h.wait()
        shandles[c] = pltpu.async_copy(
            rows.at[s],
            out_hbm.at[pl.ds(b0, BB), pl.ds(t * 2 * OUTPUT_DIM, 2 * OUTPUT_DIM)],
            ssems[s],
        )

    for c in range(PER_W + SKEW):
        if c < PER_W:
            if c >= NBUF:
                shandles[c - NBUF].wait()
            stage_v(c)
        g = c - SKEW
        if 0 <= g < PER_W:
            start_gather(g)
        d = c - SKEW - 1
        if 0 <= d < PER_W:
            start_out(d)
    start_out(PER_W - 1)
    for d in range(PER_W - NBUF, PER_W):
        shandles[d].wait()


def kernel(input_features, table):
    inT = input_features.T.reshape(N_FIELDS * BATCH)
    tbl3 = table.reshape(N_PAIRS, 2 * VALS_PER_FIELD, OUTPUT_DIM)
    left = jnp.broadcast_to(
        tbl3[:, :VALS_PER_FIELD, None, :],
        (N_PAIRS, VALS_PER_FIELD, VALS_PER_FIELD, OUTPUT_DIM),
    )
    right = jnp.broadcast_to(
        tbl3[:, None, VALS_PER_FIELD:, :],
        (N_PAIRS, VALS_PER_FIELD, VALS_PER_FIELD, OUTPUT_DIM),
    )
    ptab = jnp.concatenate([left, right], axis=3).reshape(
        N_PAIRS * VALS_PER_FIELD * VALS_PER_FIELD, 2 * OUTPUT_DIM
    )
    return _sc_gather(inT, ptab)


# speedup vs baseline: 1.0471x; 1.0000x over previous
import functools

import jax
import jax.numpy as jnp
from jax import lax
from jax.experimental import pallas as pl
from jax.experimental.pallas import tpu as pltpu
from jax.experimental.pallas import tpu_sc as plsc

N_FIELDS = 26
N_PAIRS = N_FIELDS // 2
VALS_PER_FIELD = 32
OUTPUT_DIM = 64
BATCH = 16384

_info = plsc.get_sparse_core_info()
NC, NS, L = _info.num_cores, _info.num_subcores, _info.num_lanes
NW = NC * NS
BB = 256
GI = 128
N_BCHUNK = BATCH // BB
TOT_CHUNKS = N_PAIRS * N_BCHUNK
PER_W = TOT_CHUNKS // NW
NBUF = 3
SKEW = 1


@functools.partial(
    pl.kernel,
    mesh=plsc.VectorSubcoreMesh(core_axis_name="c", subcore_axis_name="s"),
    out_type=jax.ShapeDtypeStruct((BATCH, N_FIELDS * OUTPUT_DIM), jnp.float32),
    scratch_types=[
        pltpu.VMEM((NBUF * 2 * BB,), jnp.int32),
        pltpu.VMEM((NBUF * BB,), jnp.int32),
        pltpu.VMEM((NBUF, BB, 2 * OUTPUT_DIM), jnp.float32),
    ]
    + [pltpu.SemaphoreType.DMA] * (3 * NBUF),
    compiler_params=pltpu.CompilerParams(use_tc_tiling_on_sc=True),
)
def _sc_gather(inT_hbm, ptab_hbm, out_hbm, vbuf, idxbuf, rows, *sems):
    vsems = sems[:NBUF]
    gsems = sems[NBUF : 2 * NBUF]
    ssems = sems[2 * NBUF :]
    wid = lax.axis_index("s") * NC + lax.axis_index("c")
    cid0 = wid * PER_W
    lane = lax.iota(jnp.int32, L)

    vhandles = [None] * PER_W
    ghandles = [None] * PER_W
    shandles = [None] * PER_W

    def chunk_coords(c):
        cid = cid0 + c
        t = cid // N_BCHUNK
        b0 = (cid % N_BCHUNK) * BB
        return t, b0

    def stage_v(c):
        s = c % NBUF
        t, b0 = chunk_coords(c)
        h0 = pltpu.async_copy(
            inT_hbm.at[pl.ds(2 * t * BATCH + b0, BB)],
            vbuf.at[pl.ds((s * 2) * BB, BB)],
            vsems[s],
        )
        h1 = pltpu.async_copy(
            inT_hbm.at[pl.ds((2 * t + 1) * BATCH + b0, BB)],
            vbuf.at[pl.ds((s * 2 + 1) * BB, BB)],
            vsems[s],
        )
        vhandles[c] = (h0, h1)

    def start_gather(c):
        s = c % NBUF
        t, _ = chunk_coords(c)
        for h in vhandles[c]:
            h.wait()
        for i in range(BB // L):
            v0 = vbuf[pl.ds((s * 2) * BB + i * L, L)]
            v1 = vbuf[pl.ds((s * 2 + 1) * BB + i * L, L)]
            c0 = jnp.where((v0 >= 0) & (v0 < VALS_PER_FIELD), v0, 0)
            c1 = jnp.where((v1 >= 0) & (v1 < VALS_PER_FIELD), v1, 0)
            idxbuf[pl.ds(s * BB + i * L, L)] = t * 1024 + c0 * VALS_PER_FIELD + c1
        h0 = pltpu.async_copy(
            ptab_hbm.at[idxbuf.at[pl.ds(s * BB, GI)]],
            rows.at[s, pl.ds(0, GI)],
            gsems[s],
        )
        h1 = pltpu.async_copy(
            ptab_hbm.at[idxbuf.at[pl.ds(s * BB + GI, GI)]],
            rows.at[s, pl.ds(GI, GI)],
            gsems[s],
        )
        ghandles[c] = (h0, h1)

    def start_out(c):
        s = c % NBUF
        t, b0 = chunk_coords(c)
        col = pl.ds(t * 2 * OUTPUT_DIM, 2 * OUTPUT_DIM)
        outs = []
        for h_idx, h in enumerate(ghandles[c]):
            h.wait()
            outs.append(
                pltpu.async_copy(
                    rows.at[s, pl.ds(h_idx * GI, GI)],
                    out_hbm.at[pl.ds(b0 + h_idx * GI, GI), col],
                    ssems[s],
                )
            )
        shandles[c] = outs

    for c in range(PER_W + SKEW):
        if c < PER_W:
            if c >= NBUF:
                for h in shandles[c - NBUF]:
                    h.wait()
            stage_v(c)
        g = c - SKEW
        if 0 <= g < PER_W:
            start_gather(g)
        d = c - SKEW - 1
        if 0 <= d < PER_W:
            start_out(d)
    start_out(PER_W - 1)
    for d in range(PER_W - NBUF, PER_W):
        for h in shandles[d]:
            h.wait()


def kernel(input_features, table):
    inT = input_features.T.reshape(N_FIELDS * BATCH)
    tbl3 = table.reshape(N_PAIRS, 2 * VALS_PER_FIELD, OUTPUT_DIM)
    left = jnp.broadcast_to(
        tbl3[:, :VALS_PER_FIELD, None, :],
        (N_PAIRS, VALS_PER_FIELD, VALS_PER_FIELD, OUTPUT_DIM),
    )
    right = jnp.broadcast_to(
        tbl3[:, None, VALS_PER_FIELD:, :],
        (N_PAIRS, VALS_PER_FIELD, VALS_PER_FIELD, OUTPUT_DIM),
    )
    ptab = jnp.concatenate([left, right], axis=3).reshape(
        N_PAIRS * VALS_PER_FIELD * VALS_PER_FIELD, 2 * OUTPUT_DIM
    )
    return _sc_gather(inT, ptab)
